# baseline (speedup 1.0000x reference)
import functools

import jax
import jax.numpy as jnp
from jax import lax
from jax.experimental import pallas as pl
from jax.experimental.pallas import tpu as pltpu
from jax.experimental.pallas import tpu_sc as plsc

B = 16384
D = 128
NC = 2
NS = 16
NW = NC * NS
BPW = B // NW
CHUNK = 64
NCHUNK = BPW // CHUNK
NSLOT = 4
AHEAD = 3


def _body(u_hbm, i_hbm, emb_u_hbm, emb_i_hbm, out_hbm,
          uidx, iidx, ubuf, ibuf, out_v, *sems):
    cid = lax.axis_index("c")
    sid = lax.axis_index("s")
    wid = sid * NC + cid
    base = wid * BPW

    def start(j):
        slot = j % NSLOT
        pltpu.sync_copy(u_hbm.at[pl.ds(base + j * CHUNK, CHUNK)], uidx.at[j])
        pltpu.sync_copy(i_hbm.at[pl.ds(base + j * CHUNK, CHUNK)], iidx.at[j])
        cu = pltpu.async_copy(emb_u_hbm.at[uidx.at[j]], ubuf.at[slot],
                              sems[2 * slot])
        ci = pltpu.async_copy(emb_i_hbm.at[iidx.at[j]], ibuf.at[slot],
                              sems[2 * slot + 1])
        return cu, ci

    pending = [start(j) for j in range(AHEAD)]
    for j in range(NCHUNK):
        slot = j % NSLOT
        cu, ci = pending[j]
        cu.wait()
        ci.wait()
        if j + AHEAD < NCHUNK:
            pending.append(start(j + AHEAD))

        def row(r, carry, j=j, slot=slot):
            acc = jnp.zeros((16,), jnp.float32)
            for d8 in range(D // 16):
                uv = ubuf[slot, r, pl.ds(d8 * 16, 16)]
                iv = ibuf[slot, r, pl.ds(d8 * 16, 16)]
                acc = acc + uv * iv
            tot = plsc.cumsum(acc)
            lane = lax.iota(jnp.int32, 16)
            pos = jnp.full((16,), j * CHUNK + r, jnp.int32)
            plsc.store_scatter(out_v, [pos], tot, mask=lane == 15)
            return carry

        lax.fori_loop(0, CHUNK, row, 0, unroll=4)

    pltpu.sync_copy(out_v, out_hbm.at[pl.ds(base, BPW)])


_sc_call = pl.kernel(
    _body,
    out_type=jax.ShapeDtypeStruct((B,), jnp.float32),
    mesh=plsc.VectorSubcoreMesh(
        core_axis_name="c", subcore_axis_name="s",
        num_cores=NC, num_subcores=NS),
    scratch_types=[
        pltpu.VMEM((NCHUNK, CHUNK), jnp.int32),
        pltpu.VMEM((NCHUNK, CHUNK), jnp.int32),
        pltpu.VMEM((NSLOT, CHUNK, D), jnp.float32),
        pltpu.VMEM((NSLOT, CHUNK, D), jnp.float32),
        pltpu.VMEM((BPW,), jnp.float32),
    ] + [pltpu.SemaphoreType.DMA] * (2 * NSLOT),
    compiler_params=pltpu.CompilerParams(needs_layout_passes=False),
)


@jax.jit
def kernel(u, i, emb_u, emb_i):
    return _sc_call(u.astype(jnp.int32), i.astype(jnp.int32), emb_u, emb_i)


# speedup vs baseline: 1.0937x; 1.0937x over previous
import functools

import jax
import jax.numpy as jnp
from jax import lax
from jax.experimental import pallas as pl
from jax.experimental.pallas import tpu as pltpu
from jax.experimental.pallas import tpu_sc as plsc

B = 16384
D = 128
NC = 2
NS = 16
NW = NC * NS
BPW = B // NW
CHUNK = 128
NCHUNK = BPW // CHUNK
NSLOT = 3
AHEAD = 2


def _body(u_hbm, i_hbm, emb_u_hbm, emb_i_hbm, out_hbm,
          uidx, iidx, ubuf, ibuf, out_v, *sems):
    cid = lax.axis_index("c")
    sid = lax.axis_index("s")
    wid = sid * NC + cid
    base = wid * BPW

    def start(j):
        slot = j % NSLOT
        pltpu.sync_copy(u_hbm.at[pl.ds(base + j * CHUNK, CHUNK)], uidx.at[j])
        pltpu.sync_copy(i_hbm.at[pl.ds(base + j * CHUNK, CHUNK)], iidx.at[j])
        cu = pltpu.async_copy(emb_u_hbm.at[uidx.at[j]], ubuf.at[slot],
                              sems[2 * slot])
        ci = pltpu.async_copy(emb_i_hbm.at[iidx.at[j]], ibuf.at[slot],
                              sems[2 * slot + 1])
        return cu, ci

    pending = [start(j) for j in range(AHEAD)]
    for j in range(NCHUNK):
        slot = j % NSLOT
        cu, ci = pending[j]
        cu.wait()
        ci.wait()
        if j + AHEAD < NCHUNK:
            pending.append(start(j + AHEAD))

        def row(r, carry, j=j, slot=slot):
            acc = jnp.zeros((16,), jnp.float32)
            for d8 in range(D // 16):
                uv = ubuf[slot, r, pl.ds(d8 * 16, 16)]
                iv = ibuf[slot, r, pl.ds(d8 * 16, 16)]
                acc = acc + uv * iv
            tot = plsc.cumsum(acc)
            lane = lax.iota(jnp.int32, 16)
            pos = jnp.full((16,), j * CHUNK + r, jnp.int32)
            plsc.store_scatter(out_v, [pos], tot, mask=lane == 15)
            return carry

        lax.fori_loop(0, CHUNK, row, 0, unroll=4)

    pltpu.sync_copy(out_v, out_hbm.at[pl.ds(base, BPW)])


_sc_call = pl.kernel(
    _body,
    out_type=jax.ShapeDtypeStruct((B,), jnp.float32),
    mesh=plsc.VectorSubcoreMesh(
        core_axis_name="c", subcore_axis_name="s",
        num_cores=NC, num_subcores=NS),
    scratch_types=[
        pltpu.VMEM((NCHUNK, CHUNK), jnp.int32),
        pltpu.VMEM((NCHUNK, CHUNK), jnp.int32),
        pltpu.VMEM((NSLOT, CHUNK, D), jnp.float32),
        pltpu.VMEM((NSLOT, CHUNK, D), jnp.float32),
        pltpu.VMEM((BPW,), jnp.float32),
    ] + [pltpu.SemaphoreType.DMA] * (2 * NSLOT),
    compiler_params=pltpu.CompilerParams(needs_layout_passes=False),
)


@jax.jit
def kernel(u, i, emb_u, emb_i):
    return _sc_call(u.astype(jnp.int32), i.astype(jnp.int32), emb_u, emb_i)


# speedup vs baseline: 1.0961x; 1.0022x over previous
import functools

import jax
import jax.numpy as jnp
from jax import lax
from jax.experimental import pallas as pl
from jax.experimental.pallas import tpu as pltpu
from jax.experimental.pallas import tpu_sc as plsc

B = 16384
D = 128
NC = 2
NS = 16
NW = NC * NS
BPW = B // NW
CHUNK = 128
NCHUNK = BPW // CHUNK
NSLOT = 3
AHEAD = 2


def _body(u_hbm, i_hbm, emb_u_hbm, emb_i_hbm, out_hbm,
          uidx, iidx, ubuf, ibuf, out_v, *sems):
    cid = lax.axis_index("c")
    sid = lax.axis_index("s")
    wid = sid * NC + cid
    base = wid * BPW

    def start(j):
        slot = j % NSLOT
        pltpu.sync_copy(u_hbm.at[pl.ds(base + j * CHUNK, CHUNK)], uidx.at[j])
        pltpu.sync_copy(i_hbm.at[pl.ds(base + j * CHUNK, CHUNK)], iidx.at[j])
        cu = pltpu.async_copy(emb_u_hbm.at[uidx.at[j]], ubuf.at[slot],
                              sems[2 * slot])
        ci = pltpu.async_copy(emb_i_hbm.at[iidx.at[j]], ibuf.at[slot],
                              sems[2 * slot + 1])
        return cu, ci

    pending = [start(j) for j in range(AHEAD)]
    for j in range(NCHUNK):
        slot = j % NSLOT
        cu, ci = pending[j]
        cu.wait()
        ci.wait()
        if j + AHEAD < NCHUNK:
            pending.append(start(j + AHEAD))

        def row(r, carry, j=j, slot=slot):
            acc = jnp.zeros((16,), jnp.float32)
            for d8 in range(D // 16):
                uv = ubuf[slot, r, pl.ds(d8 * 16, 16)]
                iv = ibuf[slot, r, pl.ds(d8 * 16, 16)]
                acc = acc + uv * iv
            tot = plsc.cumsum(acc)
            lane = lax.iota(jnp.int32, 16)
            pos = jnp.full((16,), j * CHUNK + r, jnp.int32)
            plsc.store_scatter(out_v, [pos], tot, mask=lane == 15)
            return carry

        lax.fori_loop(0, CHUNK, row, 0, unroll=4)

    pltpu.sync_copy(out_v, out_hbm.at[pl.ds(base, BPW)])


_sc_call = pl.kernel(
    _body,
    out_type=jax.ShapeDtypeStruct((B,), jnp.float32),
    mesh=plsc.VectorSubcoreMesh(
        core_axis_name="c", subcore_axis_name="s",
        num_cores=NC, num_subcores=NS),
    scratch_types=[
        pltpu.VMEM((NCHUNK, CHUNK), jnp.int32),
        pltpu.VMEM((NCHUNK, CHUNK), jnp.int32),
        pltpu.VMEM((NSLOT, CHUNK, D), jnp.float32),
        pltpu.VMEM((NSLOT, CHUNK, D), jnp.float32),
        pltpu.VMEM((BPW,), jnp.float32),
    ] + [pltpu.SemaphoreType.DMA] * (2 * NSLOT),
    compiler_params=pltpu.CompilerParams(needs_layout_passes=False, skip_device_barrier=True),
)


@jax.jit
def kernel(u, i, emb_u, emb_i):
    return _sc_call(u.astype(jnp.int32), i.astype(jnp.int32), emb_u, emb_i)
